# initial kernel scaffold (unmeasured)
import jax
import jax.numpy as jnp
from jax import lax
from jax.experimental import pallas as pl
from jax.experimental.pallas import tpu as pltpu


def kernel(
    x,
):
    def body(*refs):
        pass

    out_shape = jax.ShapeDtypeStruct(..., jnp.float32)
    return pl.pallas_call(body, out_shape=out_shape)(...)


# baseline (device time: 48617 ns/iter reference)
import jax
import jax.numpy as jnp
from jax import lax
from jax.experimental import pallas as pl
from jax.experimental.pallas import tpu as pltpu

N_DEV = 32
M, N = 512, 512
RS_BITS = (8, 1, 4, 16, 2)
AG_BITS = tuple(reversed(RS_BITS))
RS_HALVES = tuple((M >> (s + 1)) for s in range(5))
RS_OFFS = tuple(sum(RS_HALVES[:s]) for s in range(5))


def kernel(x):
    def body(x_ref, out_ref, rs_scratch, send_sems, recv_sems):
        my = lax.axis_index("i")

        barrier_sem = pltpu.get_barrier_semaphore()
        for b in RS_BITS:
            pl.semaphore_signal(
                barrier_sem, inc=1,
                device_id=(my ^ b,), device_id_type=pl.DeviceIdType.MESH,
            )
        pl.semaphore_wait(barrier_sem, len(RS_BITS))

        out_ref[:, :] = x_ref[:, :]

        lo = jnp.int32(0)
        for s, b in enumerate(RS_BITS):
            half = RS_HALVES[s]
            partner = my ^ b
            upper = (my & b) != 0
            send_lo = pl.multiple_of(jnp.where(upper, lo, lo + half), 16)
            keep_lo = pl.multiple_of(jnp.where(upper, lo + half, lo), 16)
            rdma = pltpu.make_async_remote_copy(
                src_ref=out_ref.at[pl.ds(send_lo, half), :],
                dst_ref=rs_scratch.at[pl.ds(RS_OFFS[s], half), :],
                send_sem=send_sems.at[s],
                recv_sem=recv_sems.at[s],
                device_id=(partner,),
                device_id_type=pl.DeviceIdType.MESH,
            )
            rdma.start()
            rdma.wait()
            out_ref[pl.ds(keep_lo, half), :] = (
                out_ref[pl.ds(keep_lo, half), :]
                + rs_scratch[pl.ds(RS_OFFS[s], half), :]
            )
            lo = keep_lo

        for t, b in enumerate(AG_BITS):
            seg = 16 << t
            lo = pl.multiple_of(lo, 16)
            partner = my ^ b
            rdma = pltpu.make_async_remote_copy(
                src_ref=out_ref.at[pl.ds(lo, seg), :],
                dst_ref=out_ref.at[pl.ds(lo, seg), :],
                send_sem=send_sems.at[5 + t],
                recv_sem=recv_sems.at[5 + t],
                device_id=(partner,),
                device_id_type=pl.DeviceIdType.MESH,
            )
            rdma.start()
            rdma.wait()
            lo = jnp.minimum(lo, jnp.where((my & b) != 0, lo - seg, lo + seg))

    return pl.pallas_call(
        body,
        out_shape=jax.ShapeDtypeStruct((M, N), x.dtype),
        in_specs=[pl.BlockSpec(memory_space=pltpu.VMEM)],
        out_specs=pl.BlockSpec(memory_space=pltpu.VMEM),
        scratch_shapes=[
            pltpu.VMEM((sum(RS_HALVES), N), x.dtype),
            pltpu.SemaphoreType.DMA((10,)),
            pltpu.SemaphoreType.DMA((10,)),
        ],
        compiler_params=pltpu.CompilerParams(collective_id=0),
    )(x)
